# baseline (speedup 1.0000x reference)
import functools

import jax
import jax.numpy as jnp
from jax import lax
from jax.experimental import pallas as pl
from jax.experimental.pallas import tpu as pltpu
from jax.experimental.pallas import tpu_sc as plsc

NC = 2
NS = 16
NW = NC * NS


@functools.partial(jax.jit, static_argnames=("b", "s", "d"))
def _gather_rows(ids, table, b, s, d):
    rows_per_w = (b * s) // NW
    w_per_b = s // rows_per_w
    chunk = 8
    nbuf = 6
    n_chunks = rows_per_w // chunk

    mesh = plsc.VectorSubcoreMesh(core_axis_name="c", subcore_axis_name="s")

    @functools.partial(
        pl.kernel,
        out_type=jax.ShapeDtypeStruct((b, s, d), jnp.float32),
        mesh=mesh,
        scratch_types=[
            pltpu.VMEM((rows_per_w,), jnp.int32),
            pltpu.VMEM((nbuf * chunk, d), jnp.float32),
            *[pltpu.SemaphoreType.DMA for _ in range(nbuf + nbuf // 2)],
        ],
    )
    def k(ids_hbm, table_hbm, out_hbm, idx_v, ring, *sems):
        gsems = sems[:nbuf]
        ssems = sems[nbuf:]
        npair = nbuf // 2
        wid = lax.axis_index("s") * NC + lax.axis_index("c")
        b_idx = wid // w_per_b
        col0 = (wid % w_per_b) * rows_per_w
        base = wid * rows_per_w

        def fire_gather(g):
            p = g % nbuf
            return pltpu.async_copy(
                table_hbm.at[idx_v.at[pl.ds(g * chunk, chunk)]],
                ring.at[pl.ds(p * chunk, chunk)],
                gsems[p],
            )

        def fire_store(j):
            p = (2 * j) % nbuf
            return pltpu.async_copy(
                ring.at[pl.ds(p * chunk, 2 * chunk)],
                out_hbm.at[b_idx, pl.ds(col0 + 2 * j * chunk, 2 * chunk)],
                ssems[j % npair],
            )

        head = (nbuf - 1) * chunk
        pltpu.sync_copy(ids_hbm.at[pl.ds(base, head)], idx_v.at[pl.ds(0, head)])
        gathers = {}
        stores = {}
        for g in range(min(nbuf - 1, n_chunks)):
            gathers[g] = fire_gather(g)
        pltpu.sync_copy(
            ids_hbm.at[pl.ds(base + head, rows_per_w - head)],
            idx_v.at[pl.ds(head, rows_per_w - head)],
        )
        n_pairs = n_chunks // 2
        for g in range(n_chunks):
            gathers[g].wait()
            if g % 2 == 1:
                stores[g // 2] = fire_store(g // 2)
            nxt = g + nbuf - 1
            if nxt < n_chunks:
                jp = (nxt - nbuf) // 2
                if jp >= 0 and (nxt - nbuf) % 2 == 0:
                    stores[jp].wait()
                gathers[nxt] = fire_gather(nxt)
        for j in range(max(0, (n_chunks - nbuf) // 2), n_pairs):
            stores[j].wait()

    return k(ids.reshape(b * s), table)


def kernel(input_ids, table):
    b, s = input_ids.shape
    v, d = table.shape
    if input_ids.dtype != jnp.int32:
        input_ids = input_ids.astype(jnp.int32)
    return _gather_rows(input_ids, table, b, s, d)


# speedup vs baseline: 1.0041x; 1.0041x over previous
import functools

import jax
import jax.numpy as jnp
from jax import lax
from jax.experimental import pallas as pl
from jax.experimental.pallas import tpu as pltpu
from jax.experimental.pallas import tpu_sc as plsc

NC = 2
NS = 16
NW = NC * NS


@functools.partial(jax.jit, static_argnames=("b", "s", "d"))
def _gather_rows(ids, table, b, s, d):
    rows_per_w = (b * s) // NW
    w_per_b = s // rows_per_w
    chunk = 8
    nbuf = 6
    n_chunks = rows_per_w // chunk

    mesh = plsc.VectorSubcoreMesh(core_axis_name="c", subcore_axis_name="s")

    @functools.partial(
        pl.kernel,
        out_type=jax.ShapeDtypeStruct((b, s, d), jnp.float32),
        mesh=mesh,
        scratch_types=[
            pltpu.VMEM((rows_per_w,), jnp.int32),
            *[pltpu.VMEM((chunk, d), jnp.float32) for _ in range(nbuf)],
            *[pltpu.SemaphoreType.DMA for _ in range(2 * nbuf)],
        ],
    )
    def k(ids_hbm, table_hbm, out_hbm, idx_v, *scr):
        bufs = scr[:nbuf]
        gsems = scr[nbuf : 2 * nbuf]
        ssems = scr[2 * nbuf :]
        wid = lax.axis_index("s") * NC + lax.axis_index("c")
        b_idx = wid // w_per_b
        col0 = (wid % w_per_b) * rows_per_w
        base = wid * rows_per_w
        def fire_gather(g):
            p = g % nbuf
            return pltpu.async_copy(
                table_hbm.at[idx_v.at[pl.ds(g * chunk, chunk)]], bufs[p], gsems[p]
            )

        head = (nbuf - 1) * chunk
        pltpu.sync_copy(ids_hbm.at[pl.ds(base, head)], idx_v.at[pl.ds(0, head)])
        gathers = {}
        stores = {}
        for g in range(min(nbuf - 1, n_chunks)):
            gathers[g] = fire_gather(g)
        pltpu.sync_copy(
            ids_hbm.at[pl.ds(base + head, rows_per_w - head)],
            idx_v.at[pl.ds(head, rows_per_w - head)],
        )
        for g in range(n_chunks):
            p = g % nbuf
            gathers[g].wait()
            stores[g] = pltpu.async_copy(
                bufs[p], out_hbm.at[b_idx, pl.ds(col0 + g * chunk, chunk)], ssems[p]
            )
            nxt = g + nbuf - 1
            if nxt < n_chunks:
                if g >= 1:
                    stores[g - 1].wait()
                gathers[nxt] = fire_gather(nxt)
        for g in range(max(0, n_chunks - nbuf), n_chunks):
            stores[g].wait()

    return k(ids.reshape(b * s), table)


def kernel(input_ids, table):
    b, s = input_ids.shape
    v, d = table.shape
    if input_ids.dtype != jnp.int32:
        input_ids = input_ids.astype(jnp.int32)
    return _gather_rows(input_ids, table, b, s, d)
